# Initial kernel scaffold; baseline (speedup 1.0000x reference)
import jax
import jax.numpy as jnp
from jax.experimental import pallas as pl


def kernel(x, edge_index, W1, b1, W2, b2):
    raise NotImplementedError("write your pallas kernel here")


# speedup vs baseline: 14.6968x; 14.6968x over previous
import functools

import jax
import jax.numpy as jnp
from jax import lax
from jax.experimental import pallas as pl
from jax.experimental.pallas import tpu as pltpu
from jax.experimental.pallas import tpu_sc as plsc

NC = 2
NS = 16
NW = NC * NS
CH = 128

HID = 64
DEGW = 16


def _sc_mesh():
  return plsc.VectorSubcoreMesh(
      core_axis_name="c", subcore_axis_name="s", num_cores=NC, num_subcores=NS
  )


def _make_deg_kernel(n_pad, ept):
  kch = ept // CH
  rpw = n_pad // NS

  @functools.partial(
      pl.kernel,
      mesh=_sc_mesh(),
      compiler_params=pltpu.CompilerParams(use_tc_tiling_on_sc=False),
      out_type=jax.ShapeDtypeStruct((NC, n_pad, DEGW), jnp.float32),
      scratch_types=[
          pltpu.VMEM((CH,), jnp.int32),
          pltpu.VMEM((CH, DEGW), jnp.float32),
          pltpu.VMEM_SHARED((n_pad, DEGW), jnp.float32),
      ],
  )
  def deg_kernel(dst_hbm, zeros_hbm, ones_hbm, out_hbm, dst_v, ones_v, acc):
    c = lax.axis_index("c")
    s = lax.axis_index("s")
    wid = c * NS + s
    pltpu.sync_copy(ones_hbm, ones_v)
    pltpu.sync_copy(
        zeros_hbm.at[pl.ds(s * rpw, rpw)], acc.at[pl.ds(s * rpw, rpw)]
    )
    plsc.subcore_barrier()

    def body(k, carry):
      base = wid * ept + k * CH
      pltpu.sync_copy(dst_hbm.at[pl.ds(base, CH)], dst_v)
      pltpu.sync_copy(ones_v, acc.at[dst_v], add=True)
      return carry

    lax.fori_loop(0, kch, body, 0)
    plsc.subcore_barrier()
    pltpu.sync_copy(
        acc.at[pl.ds(s * rpw, rpw)], out_hbm.at[c].at[pl.ds(s * rpw, rpw)]
    )

  return deg_kernel


def _make_agg_kernel(n_pad, ept):
  kch = ept // CH
  rpw = n_pad // NS

  @functools.partial(
      pl.kernel,
      mesh=_sc_mesh(),
      compiler_params=pltpu.CompilerParams(use_tc_tiling_on_sc=False),
      out_type=jax.ShapeDtypeStruct((NC, n_pad, HID), jnp.float32),
      scratch_types=[
          pltpu.VMEM((CH,), jnp.int32),
          pltpu.VMEM((CH,), jnp.int32),
          pltpu.VMEM((CH, HID), jnp.float32),
          pltpu.VMEM_SHARED((n_pad, HID), jnp.float32),
          pltpu.SemaphoreType.DMA,
      ],
  )
  def agg_kernel(hs_hbm, src_hbm, dst_hbm, out_hbm, src_v, dst_v, rows_v, acc,
                 gsem):
    c = lax.axis_index("c")
    s = lax.axis_index("s")
    wid = c * NS + s
    pltpu.sync_copy(
        hs_hbm.at[pl.ds(s * rpw, rpw)], acc.at[pl.ds(s * rpw, rpw)]
    )
    plsc.subcore_barrier()

    def body(k, carry):
      base = wid * ept + k * CH
      pltpu.sync_copy(src_hbm.at[pl.ds(base, CH)], src_v)
      pltpu.sync_copy(dst_hbm.at[pl.ds(base, CH)], dst_v)
      pltpu.async_copy(hs_hbm.at[src_v], rows_v, gsem).wait()
      pltpu.sync_copy(rows_v, acc.at[dst_v], add=True)
      return carry

    lax.fori_loop(0, kch, body, 0)
    plsc.subcore_barrier()
    pltpu.sync_copy(
        acc.at[pl.ds(s * rpw, rpw)], out_hbm.at[c].at[pl.ds(s * rpw, rpw)]
    )

  return agg_kernel


def _dis_from_parts(dp_blk):
  deg = dp_blk[0, :, :1] + dp_blk[1, :, :1] + 1.0
  return lax.rsqrt(deg)


def _tc1_body(x_ref, w1_ref, dp_ref, hs_ref):
  dis = _dis_from_parts(dp_ref[...])
  h = jnp.dot(x_ref[...], w1_ref[...], preferred_element_type=jnp.float32)
  hs_ref[...] = h * dis


def _tc2_body(agg_ref, hs1_ref, dp_ref, b1_ref, hs2_ref):
  dis = _dis_from_parts(dp_ref[...])
  a = agg_ref[0] + agg_ref[1] - hs1_ref[...]
  t = jnp.maximum(a * dis + b1_ref[...], 0.0)
  hs2_ref[...] = t * dis


def _tc3_body(agg_ref, hs2_ref, dp_ref, w2_ref, b2_ref, out_ref, *, out_dim):
  dis = _dis_from_parts(dp_ref[...])
  u = (agg_ref[0] + agg_ref[1] - hs2_ref[...]) * dis
  h2 = jnp.dot(u, w2_ref[...], preferred_element_type=jnp.float32)
  h2 = h2 + b2_ref[...]
  col = lax.broadcasted_iota(jnp.int32, h2.shape, 1)
  h2m = jnp.where(col < out_dim, h2, -jnp.inf)
  m = jnp.max(h2m, axis=1, keepdims=True)
  lse = m + jnp.log(jnp.sum(jnp.exp(h2m - m), axis=1, keepdims=True))
  out_ref[...] = h2 - lse


def kernel(x, edge_index, W1, b1, W2, b2):
  n, in_dim = x.shape
  e = edge_index.shape[1]
  hid = W1.shape[1]
  out_dim = W2.shape[1]

  br = 512
  n_pad = ((n + 1 + br - 1) // br) * br
  e_pad = ((e + NW * CH - 1) // (NW * CH)) * (NW * CH)
  ept = e_pad // NW
  grid = n_pad // br

  ei = edge_index.astype(jnp.int32)
  pad_idx = jnp.full((e_pad - e,), n, jnp.int32)
  src_p = jnp.concatenate([ei[0], pad_idx])
  dst_p = jnp.concatenate([ei[1], pad_idx])
  x_p = jnp.concatenate([x, jnp.zeros((n_pad - n, in_dim), x.dtype)])

  deg_kernel = _make_deg_kernel(n_pad, ept)
  deg_parts = deg_kernel(
      dst_p, jnp.zeros((n_pad, DEGW), jnp.float32),
      jnp.ones((CH, DEGW), jnp.float32)
  )

  hs1 = pl.pallas_call(
      _tc1_body,
      grid=(grid,),
      in_specs=[
          pl.BlockSpec((br, in_dim), lambda i: (i, 0)),
          pl.BlockSpec((in_dim, hid), lambda i: (0, 0)),
          pl.BlockSpec((NC, br, DEGW), lambda i: (0, i, 0)),
      ],
      out_specs=pl.BlockSpec((br, hid), lambda i: (i, 0)),
      out_shape=jax.ShapeDtypeStruct((n_pad, hid), jnp.float32),
  )(x_p, W1, deg_parts)

  agg_kernel = _make_agg_kernel(n_pad, ept)

  agg1 = agg_kernel(hs1, src_p, dst_p)

  hs2 = pl.pallas_call(
      _tc2_body,
      grid=(grid,),
      in_specs=[
          pl.BlockSpec((NC, br, hid), lambda i: (0, i, 0)),
          pl.BlockSpec((br, hid), lambda i: (i, 0)),
          pl.BlockSpec((NC, br, DEGW), lambda i: (0, i, 0)),
          pl.BlockSpec((1, hid), lambda i: (0, 0)),
      ],
      out_specs=pl.BlockSpec((br, hid), lambda i: (i, 0)),
      out_shape=jax.ShapeDtypeStruct((n_pad, hid), jnp.float32),
  )(agg1, hs1, deg_parts, b1.reshape(1, hid))

  agg2 = agg_kernel(hs2, src_p, dst_p)

  ow = 128
  w2_p = jnp.zeros((hid, ow), jnp.float32).at[:, :out_dim].set(W2)
  b2_p = jnp.zeros((1, ow), jnp.float32).at[0, :out_dim].set(b2)
  out = pl.pallas_call(
      functools.partial(_tc3_body, out_dim=out_dim),
      grid=(grid,),
      in_specs=[
          pl.BlockSpec((NC, br, hid), lambda i: (0, i, 0)),
          pl.BlockSpec((br, hid), lambda i: (i, 0)),
          pl.BlockSpec((NC, br, DEGW), lambda i: (0, i, 0)),
          pl.BlockSpec((hid, ow), lambda i: (0, 0)),
          pl.BlockSpec((1, ow), lambda i: (0, 0)),
      ],
      out_specs=pl.BlockSpec((br, ow), lambda i: (i, 0)),
      out_shape=jax.ShapeDtypeStruct((n_pad, ow), jnp.float32),
  )(agg2, hs2, deg_parts, w2_p, b2_p)

  return out[:n, :out_dim]


# speedup vs baseline: 31.3118x; 2.1305x over previous
import functools

import jax
import jax.numpy as jnp
from jax import lax
from jax.experimental import pallas as pl
from jax.experimental.pallas import tpu as pltpu
from jax.experimental.pallas import tpu_sc as plsc

NC = 2
NS = 16
NW = NC * NS
CH = 128

HID = 64
DEGW = 16


def _sc_mesh():
  return plsc.VectorSubcoreMesh(
      core_axis_name="c", subcore_axis_name="s", num_cores=NC, num_subcores=NS
  )


def _make_deg_kernel(n_pad, ept):
  kch = ept // CH
  rpw = n_pad // NS

  @functools.partial(
      pl.kernel,
      mesh=_sc_mesh(),
      compiler_params=pltpu.CompilerParams(use_tc_tiling_on_sc=False),
      out_type=jax.ShapeDtypeStruct((NC, n_pad, DEGW), jnp.float32),
      scratch_types=[
          pltpu.VMEM((kch, CH), jnp.int32),
          pltpu.VMEM((CH, DEGW), jnp.float32),
          pltpu.VMEM_SHARED((n_pad, DEGW), jnp.float32),
      ],
  )
  def deg_kernel(dst_hbm, zeros_hbm, ones_hbm, out_hbm, dst_v, ones_v, acc):
    c = lax.axis_index("c")
    s = lax.axis_index("s")
    wid = c * NS + s
    pltpu.sync_copy(dst_hbm.at[pl.ds(wid * kch, kch)], dst_v)
    pltpu.sync_copy(ones_hbm, ones_v)
    pltpu.sync_copy(
        zeros_hbm.at[pl.ds(s * rpw, rpw)], acc.at[pl.ds(s * rpw, rpw)]
    )
    plsc.subcore_barrier()

    def body(k, carry):
      pltpu.sync_copy(ones_v, acc.at[dst_v.at[k]], add=True)
      return carry

    lax.fori_loop(0, kch, body, 0)
    plsc.subcore_barrier()
    pltpu.sync_copy(
        acc.at[pl.ds(s * rpw, rpw)], out_hbm.at[c].at[pl.ds(s * rpw, rpw)]
    )

  return deg_kernel


def _make_agg_kernel(n_pad, ept):
  kch = ept // CH
  rpw = n_pad // NS

  @functools.partial(
      pl.kernel,
      mesh=_sc_mesh(),
      compiler_params=pltpu.CompilerParams(use_tc_tiling_on_sc=False),
      out_type=jax.ShapeDtypeStruct((NC, n_pad, HID), jnp.float32),
      scratch_types=[
          pltpu.VMEM((kch, CH), jnp.int32),
          pltpu.VMEM((kch, CH), jnp.int32),
          pltpu.VMEM((CH, HID), jnp.float32),
          pltpu.VMEM((CH, HID), jnp.float32),
          pltpu.VMEM_SHARED((n_pad, HID), jnp.float32),
          pltpu.VMEM_SHARED((n_pad, HID), jnp.float32),
          pltpu.SemaphoreType.DMA,
          pltpu.SemaphoreType.DMA,
      ],
  )
  def agg_kernel(hs_hbm, src_hbm, dst_hbm, out_hbm, src_v, dst_v, r0, r1,
                 hst, acc, s0, s1):
    c = lax.axis_index("c")
    s = lax.axis_index("s")
    wid = c * NS + s
    pltpu.sync_copy(src_hbm.at[pl.ds(wid * kch, kch)], src_v)
    pltpu.sync_copy(dst_hbm.at[pl.ds(wid * kch, kch)], dst_v)
    pltpu.sync_copy(hs_hbm.at[pl.ds(s * rpw, rpw)], hst.at[pl.ds(s * rpw, rpw)])
    pltpu.sync_copy(hs_hbm.at[pl.ds(s * rpw, rpw)], acc.at[pl.ds(s * rpw, rpw)])
    plsc.subcore_barrier()

    pltpu.async_copy(hst.at[src_v.at[0]], r0, s0)
    pltpu.async_copy(hst.at[src_v.at[1]], r1, s1)

    def body(j, carry):
      k = 2 * j
      pltpu.make_async_copy(hst.at[src_v.at[k]], r0, s0).wait()
      pltpu.sync_copy(r0, acc.at[dst_v.at[k]], add=True)
      pltpu.async_copy(hst.at[src_v.at[k + 2]], r0, s0)
      pltpu.make_async_copy(hst.at[src_v.at[k + 1]], r1, s1).wait()
      pltpu.sync_copy(r1, acc.at[dst_v.at[k + 1]], add=True)
      pltpu.async_copy(hst.at[src_v.at[k + 3]], r1, s1)
      return carry

    lax.fori_loop(0, kch // 2 - 1, body, 0)
    ke = kch - 2
    pltpu.make_async_copy(hst.at[src_v.at[ke]], r0, s0).wait()
    pltpu.sync_copy(r0, acc.at[dst_v.at[ke]], add=True)
    pltpu.make_async_copy(hst.at[src_v.at[ke + 1]], r1, s1).wait()
    pltpu.sync_copy(r1, acc.at[dst_v.at[ke + 1]], add=True)

    plsc.subcore_barrier()
    pltpu.sync_copy(
        acc.at[pl.ds(s * rpw, rpw)], out_hbm.at[c].at[pl.ds(s * rpw, rpw)]
    )

  return agg_kernel


def _dis_from_parts(dp_blk):
  deg = dp_blk[0, :, :1] + dp_blk[1, :, :1] + 1.0
  return lax.rsqrt(deg)


def _tc1_body(x_ref, w1_ref, dp_ref, hs_ref):
  dis = _dis_from_parts(dp_ref[...])
  h = jnp.dot(x_ref[...], w1_ref[...], preferred_element_type=jnp.float32)
  hs_ref[...] = h * dis


def _tc2_body(agg_ref, hs1_ref, dp_ref, b1_ref, hs2_ref):
  dis = _dis_from_parts(dp_ref[...])
  a = agg_ref[0] + agg_ref[1] - hs1_ref[...]
  t = jnp.maximum(a * dis + b1_ref[...], 0.0)
  hs2_ref[...] = t * dis


def _tc3_body(agg_ref, hs2_ref, dp_ref, w2_ref, b2_ref, out_ref, *, out_dim):
  dis = _dis_from_parts(dp_ref[...])
  u = (agg_ref[0] + agg_ref[1] - hs2_ref[...]) * dis
  h2 = jnp.dot(u, w2_ref[...], preferred_element_type=jnp.float32)
  h2 = h2 + b2_ref[...]
  col = lax.broadcasted_iota(jnp.int32, h2.shape, 1)
  h2m = jnp.where(col < out_dim, h2, -jnp.inf)
  m = jnp.max(h2m, axis=1, keepdims=True)
  lse = m + jnp.log(jnp.sum(jnp.exp(h2m - m), axis=1, keepdims=True))
  out_ref[...] = h2 - lse


def kernel(x, edge_index, W1, b1, W2, b2):
  n, in_dim = x.shape
  e = edge_index.shape[1]
  hid = W1.shape[1]
  out_dim = W2.shape[1]

  br = 512
  n_pad = ((n + 1 + br - 1) // br) * br
  epg = NW * CH * 2
  e_pad = ((e + epg - 1) // epg) * epg
  ept = e_pad // NW
  grid = n_pad // br

  ei = edge_index.astype(jnp.int32)
  pad_idx = jnp.full((e_pad - e,), n, jnp.int32)
  src_p = jnp.concatenate([ei[0], pad_idx]).reshape(e_pad // CH, CH)
  dst_p = jnp.concatenate([ei[1], pad_idx]).reshape(e_pad // CH, CH)
  x_p = jnp.concatenate([x, jnp.zeros((n_pad - n, in_dim), x.dtype)])

  deg_kernel = _make_deg_kernel(n_pad, ept)
  deg_parts = deg_kernel(
      dst_p, jnp.zeros((n_pad, DEGW), jnp.float32),
      jnp.ones((CH, DEGW), jnp.float32)
  )

  hs1 = pl.pallas_call(
      _tc1_body,
      grid=(grid,),
      in_specs=[
          pl.BlockSpec((br, in_dim), lambda i: (i, 0)),
          pl.BlockSpec((in_dim, hid), lambda i: (0, 0)),
          pl.BlockSpec((NC, br, DEGW), lambda i: (0, i, 0)),
      ],
      out_specs=pl.BlockSpec((br, hid), lambda i: (i, 0)),
      out_shape=jax.ShapeDtypeStruct((n_pad, hid), jnp.float32),
  )(x_p, W1, deg_parts)

  agg_kernel = _make_agg_kernel(n_pad, ept)

  agg1 = agg_kernel(hs1, src_p, dst_p)

  hs2 = pl.pallas_call(
      _tc2_body,
      grid=(grid,),
      in_specs=[
          pl.BlockSpec((NC, br, hid), lambda i: (0, i, 0)),
          pl.BlockSpec((br, hid), lambda i: (i, 0)),
          pl.BlockSpec((NC, br, DEGW), lambda i: (0, i, 0)),
          pl.BlockSpec((1, hid), lambda i: (0, 0)),
      ],
      out_specs=pl.BlockSpec((br, hid), lambda i: (i, 0)),
      out_shape=jax.ShapeDtypeStruct((n_pad, hid), jnp.float32),
  )(agg1, hs1, deg_parts, b1.reshape(1, hid))

  agg2 = agg_kernel(hs2, src_p, dst_p)

  ow = 128
  w2_p = jnp.zeros((hid, ow), jnp.float32).at[:, :out_dim].set(W2)
  b2_p = jnp.zeros((1, ow), jnp.float32).at[0, :out_dim].set(b2)
  out = pl.pallas_call(
      functools.partial(_tc3_body, out_dim=out_dim),
      grid=(grid,),
      in_specs=[
          pl.BlockSpec((NC, br, hid), lambda i: (0, i, 0)),
          pl.BlockSpec((br, hid), lambda i: (i, 0)),
          pl.BlockSpec((NC, br, DEGW), lambda i: (0, i, 0)),
          pl.BlockSpec((hid, ow), lambda i: (0, 0)),
          pl.BlockSpec((1, ow), lambda i: (0, 0)),
      ],
      out_specs=pl.BlockSpec((br, ow), lambda i: (i, 0)),
      out_shape=jax.ShapeDtypeStruct((n_pad, ow), jnp.float32),
  )(agg2, hs2, deg_parts, w2_p, b2_p)

  return out[:n, :out_dim]
